# baseline (device time: 23275 ns/iter reference)
import jax
import jax.numpy as jnp
from jax import lax
from jax.experimental import pallas as pl
from jax.experimental.pallas import tpu as pltpu

N_DEV = 8
B, SQ, SKV, D_MODEL = 2, 256, 256, 512
H_PER = 4
DH = 64
WINDOW = 128
ROWS = B * SQ
CHUNK = ROWS // N_DEV
CPB = SQ // CHUNK


def kernel(x, Wq, K_ext, V_ext, Wo):
    my = lax.axis_index("i")

    K = lax.dynamic_slice_in_dim(K_ext, my * H_PER, H_PER, axis=2)
    V = lax.dynamic_slice_in_dim(V_ext, my * H_PER, H_PER, axis=2)
    K = jnp.transpose(K, (0, 2, 1, 3)).astype(jnp.bfloat16)
    V = jnp.transpose(V, (0, 2, 1, 3)).astype(jnp.bfloat16)

    def body(x_ref, wq_ref, k_ref, v_ref, wo_ref, out_ref,
             pbuf, rs_buf, ag_buf, red_ref, redb_ref,
             ss1, rs1, ss2, rs2):
        pos = lax.axis_index("i")

        barrier_sem = pltpu.get_barrier_semaphore()
        for q in range(1, N_DEV):
            pl.semaphore_signal(
                barrier_sem, inc=1,
                device_id=((pos + q) % N_DEV,),
                device_id_type=pl.DeviceIdType.MESH,
            )
        pl.semaphore_wait(barrier_sem, N_DEV - 1)

        qi = lax.broadcasted_iota(jnp.int32, (SQ, SKV), 0)
        ki = lax.broadcasted_iota(jnp.int32, (SQ, SKV), 1)
        mask = jnp.abs(qi - ki) <= WINDOW

        def send_chunk(c):
            return pltpu.make_async_remote_copy(
                src_ref=pbuf.at[pl.ds(c * CHUNK, CHUNK), :],
                dst_ref=rs_buf.at[pos],
                send_sem=ss1.at[c],
                recv_sem=rs1.at[pos],
                device_id=(c,),
                device_id_type=pl.DeviceIdType.MESH,
            )

        wq_b = wq_ref[...].astype(jnp.bfloat16)
        wo_b = wo_ref[...].astype(jnp.bfloat16)
        for b in range(B):
            q = jnp.dot(x_ref[b].astype(jnp.bfloat16), wq_b,
                        preferred_element_type=jnp.float32)
            ctx_cols = []
            for h in range(H_PER):
                qh = (q[:, DH * h:DH * (h + 1)] * 0.125).astype(jnp.bfloat16)
                s = lax.dot_general(
                    qh, k_ref[b, h],
                    (((1,), (1,)), ((), ())),
                    preferred_element_type=jnp.float32)
                s = jnp.where(mask, s, -1e9)
                m = jnp.max(s, axis=-1, keepdims=True)
                w = jnp.exp(s - m)
                w = w / jnp.sum(w, axis=-1, keepdims=True)
                ctx_cols.append(
                    jnp.dot(w.astype(jnp.bfloat16), v_ref[b, h],
                            preferred_element_type=jnp.float32))
            ctx = jnp.concatenate(ctx_cols, axis=1)
            partial = jnp.dot(ctx.astype(jnp.bfloat16), wo_b,
                              preferred_element_type=jnp.float32)
            pbuf[SQ * b:SQ * (b + 1), :] = partial.astype(jnp.bfloat16)
            for c in range(CPB * b, CPB * (b + 1)):
                @pl.when(c != pos)
                def _(c=c):
                    send_chunk(c).start()

        red_ref[...] = pbuf[pl.ds(pos * CHUNK, CHUNK), :].astype(jnp.float32)
        for s in range(N_DEV):
            @pl.when(s != pos)
            def _(s=s):
                recv = pltpu.make_async_remote_copy(
                    src_ref=pbuf.at[pl.ds(0, CHUNK), :],
                    dst_ref=rs_buf.at[s],
                    send_sem=ss1.at[s],
                    recv_sem=rs1.at[s],
                    device_id=(s,),
                    device_id_type=pl.DeviceIdType.MESH,
                )
                recv.wait_recv()
                red_ref[...] += rs_buf[s].astype(jnp.float32)
        redb_ref[...] = red_ref[...].astype(jnp.bfloat16)

        for t in range(N_DEV):
            @pl.when(t != pos)
            def _(t=t):
                pltpu.make_async_remote_copy(
                    src_ref=redb_ref,
                    dst_ref=ag_buf.at[pos],
                    send_sem=ss2.at[t],
                    recv_sem=rs2.at[pos],
                    device_id=(t,),
                    device_id_type=pl.DeviceIdType.MESH,
                ).start()

        for c in range(N_DEV):
            @pl.when(c != pos)
            def _(c=c):
                send_chunk(c).wait_send()

        out_ref[pos // CPB, pl.ds((pos % CPB) * CHUNK, CHUNK), :] = (
            red_ref[...])

        for s in range(N_DEV):
            @pl.when(s != pos)
            def _(s=s):
                recv = pltpu.make_async_remote_copy(
                    src_ref=redb_ref,
                    dst_ref=ag_buf.at[s],
                    send_sem=ss2.at[s],
                    recv_sem=rs2.at[s],
                    device_id=(s,),
                    device_id_type=pl.DeviceIdType.MESH,
                )
                recv.wait_recv()
                out_ref[s // CPB,
                        (s % CPB) * CHUNK:(s % CPB + 1) * CHUNK, :] = (
                    ag_buf[s].astype(jnp.float32))
        for t in range(N_DEV):
            @pl.when(t != pos)
            def _(t=t):
                pltpu.make_async_remote_copy(
                    src_ref=redb_ref,
                    dst_ref=ag_buf.at[pos],
                    send_sem=ss2.at[t],
                    recv_sem=rs2.at[pos],
                    device_id=(t,),
                    device_id_type=pl.DeviceIdType.MESH,
                ).wait_send()

    return pl.pallas_call(
        body,
        out_shape=jax.ShapeDtypeStruct((B, SQ, D_MODEL), jnp.float32),
        in_specs=[pl.BlockSpec(memory_space=pltpu.VMEM)] * 5,
        out_specs=pl.BlockSpec(memory_space=pltpu.VMEM),
        scratch_shapes=[
            pltpu.VMEM((ROWS, D_MODEL), jnp.bfloat16),
            pltpu.VMEM((N_DEV, CHUNK, D_MODEL), jnp.bfloat16),
            pltpu.VMEM((N_DEV, CHUNK, D_MODEL), jnp.bfloat16),
            pltpu.VMEM((CHUNK, D_MODEL), jnp.float32),
            pltpu.VMEM((CHUNK, D_MODEL), jnp.bfloat16),
            pltpu.SemaphoreType.DMA((N_DEV,)),
            pltpu.SemaphoreType.DMA((N_DEV,)),
            pltpu.SemaphoreType.DMA((N_DEV,)),
            pltpu.SemaphoreType.DMA((N_DEV,)),
        ],
        compiler_params=pltpu.CompilerParams(collective_id=0),
    )(x, Wq, K, V, Wo)


# device time: 22236 ns/iter; 1.0467x vs baseline; 1.0467x over previous
import jax
import jax.numpy as jnp
from jax import lax
from jax.experimental import pallas as pl
from jax.experimental.pallas import tpu as pltpu

N_DEV = 8
B, SQ, SKV, D_MODEL = 2, 256, 256, 512
H_PER = 4
DH = 64
WINDOW = 128
ROWS = B * SQ
CHUNK = ROWS // N_DEV
CPB = SQ // CHUNK


def kernel(x, Wq, K_ext, V_ext, Wo):
    my = lax.axis_index("i")

    K = lax.dynamic_slice_in_dim(K_ext, my * H_PER, H_PER, axis=2)
    V = lax.dynamic_slice_in_dim(V_ext, my * H_PER, H_PER, axis=2)
    K = jnp.transpose(K, (0, 2, 1, 3)).astype(jnp.bfloat16)
    V = jnp.transpose(V, (0, 2, 1, 3)).astype(jnp.bfloat16)

    xb = x.astype(jnp.bfloat16)
    Wqb = Wq.astype(jnp.bfloat16)
    Wob = Wo.astype(jnp.bfloat16)

    def body(x_ref, wq_ref, k_ref, v_ref, wo_ref, out_ref,
             pbuf, rs_buf, ag_buf, red_ref, redb_ref,
             ss1, rs1, ss2, rs2):
        pos = lax.axis_index("i")

        barrier_sem = pltpu.get_barrier_semaphore()
        for q in range(1, N_DEV):
            pl.semaphore_signal(
                barrier_sem, inc=1,
                device_id=((pos + q) % N_DEV,),
                device_id_type=pl.DeviceIdType.MESH,
            )
        pl.semaphore_wait(barrier_sem, N_DEV - 1)

        qi = lax.broadcasted_iota(jnp.int32, (SQ, SKV), 0)
        ki = lax.broadcasted_iota(jnp.int32, (SQ, SKV), 1)
        mask = jnp.abs(qi - ki) <= WINDOW

        def send_chunk(c):
            return pltpu.make_async_remote_copy(
                src_ref=pbuf.at[pl.ds(c * CHUNK, CHUNK), :],
                dst_ref=rs_buf.at[pos],
                send_sem=ss1.at[c],
                recv_sem=rs1.at[pos],
                device_id=(c,),
                device_id_type=pl.DeviceIdType.MESH,
            )

        for b in range(B):
            q = jnp.dot(x_ref[b], wq_ref[...],
                        preferred_element_type=jnp.float32)
            ctx_cols = []
            for h in range(H_PER):
                qh = (q[:, DH * h:DH * (h + 1)] * 0.125).astype(jnp.bfloat16)
                s = lax.dot_general(
                    qh, k_ref[b, h],
                    (((1,), (1,)), ((), ())),
                    preferred_element_type=jnp.float32)
                s = jnp.where(mask, s, -1e9)
                m = jnp.max(s, axis=-1, keepdims=True)
                w = jnp.exp(s - m)
                w = w / jnp.sum(w, axis=-1, keepdims=True)
                ctx_cols.append(
                    jnp.dot(w.astype(jnp.bfloat16), v_ref[b, h],
                            preferred_element_type=jnp.float32))
            ctx = jnp.concatenate(ctx_cols, axis=1)
            partial = jnp.dot(ctx.astype(jnp.bfloat16), wo_ref[...],
                              preferred_element_type=jnp.float32)
            pbuf[SQ * b:SQ * (b + 1), :] = partial.astype(jnp.bfloat16)
            for c in range(CPB * b, CPB * (b + 1)):
                @pl.when(c != pos)
                def _(c=c):
                    send_chunk(c).start()

        red_ref[...] = pbuf[pl.ds(pos * CHUNK, CHUNK), :].astype(jnp.float32)
        for s in range(N_DEV):
            @pl.when(s != pos)
            def _(s=s):
                recv = pltpu.make_async_remote_copy(
                    src_ref=pbuf.at[pl.ds(0, CHUNK), :],
                    dst_ref=rs_buf.at[s],
                    send_sem=ss1.at[s],
                    recv_sem=rs1.at[s],
                    device_id=(s,),
                    device_id_type=pl.DeviceIdType.MESH,
                )
                recv.wait_recv()
                red_ref[...] += rs_buf[s].astype(jnp.float32)
        redb_ref[...] = red_ref[...].astype(jnp.bfloat16)

        for t in range(N_DEV):
            @pl.when(t != pos)
            def _(t=t):
                pltpu.make_async_remote_copy(
                    src_ref=redb_ref,
                    dst_ref=ag_buf.at[pos],
                    send_sem=ss2.at[t],
                    recv_sem=rs2.at[pos],
                    device_id=(t,),
                    device_id_type=pl.DeviceIdType.MESH,
                ).start()

        for c in range(N_DEV):
            @pl.when(c != pos)
            def _(c=c):
                send_chunk(c).wait_send()

        out_ref[pos // CPB, pl.ds((pos % CPB) * CHUNK, CHUNK), :] = (
            red_ref[...])

        for s in range(N_DEV):
            @pl.when(s != pos)
            def _(s=s):
                recv = pltpu.make_async_remote_copy(
                    src_ref=redb_ref,
                    dst_ref=ag_buf.at[s],
                    send_sem=ss2.at[s],
                    recv_sem=rs2.at[s],
                    device_id=(s,),
                    device_id_type=pl.DeviceIdType.MESH,
                )
                recv.wait_recv()
                out_ref[s // CPB,
                        (s % CPB) * CHUNK:(s % CPB + 1) * CHUNK, :] = (
                    ag_buf[s].astype(jnp.float32))
        for t in range(N_DEV):
            @pl.when(t != pos)
            def _(t=t):
                pltpu.make_async_remote_copy(
                    src_ref=redb_ref,
                    dst_ref=ag_buf.at[pos],
                    send_sem=ss2.at[t],
                    recv_sem=rs2.at[pos],
                    device_id=(t,),
                    device_id_type=pl.DeviceIdType.MESH,
                ).wait_send()

    return pl.pallas_call(
        body,
        out_shape=jax.ShapeDtypeStruct((B, SQ, D_MODEL), jnp.float32),
        in_specs=[pl.BlockSpec(memory_space=pltpu.VMEM)] * 5,
        out_specs=pl.BlockSpec(memory_space=pltpu.VMEM),
        scratch_shapes=[
            pltpu.VMEM((ROWS, D_MODEL), jnp.bfloat16),
            pltpu.VMEM((N_DEV, CHUNK, D_MODEL), jnp.bfloat16),
            pltpu.VMEM((N_DEV, CHUNK, D_MODEL), jnp.bfloat16),
            pltpu.VMEM((CHUNK, D_MODEL), jnp.float32),
            pltpu.VMEM((CHUNK, D_MODEL), jnp.bfloat16),
            pltpu.SemaphoreType.DMA((N_DEV,)),
            pltpu.SemaphoreType.DMA((N_DEV,)),
            pltpu.SemaphoreType.DMA((N_DEV,)),
            pltpu.SemaphoreType.DMA((N_DEV,)),
        ],
        compiler_params=pltpu.CompilerParams(collective_id=0),
    )(xb, Wqb, K, V, Wob)


# device time: 21908 ns/iter; 1.0624x vs baseline; 1.0150x over previous
import jax
import jax.numpy as jnp
from jax import lax
from jax.experimental import pallas as pl
from jax.experimental.pallas import tpu as pltpu

N_DEV = 8
B, SQ, SKV, D_MODEL = 2, 256, 256, 512
H_PER = 4
DH = 64
WINDOW = 128
ROWS = B * SQ
CHUNK = ROWS // N_DEV
CPB = SQ // CHUNK


def kernel(x, Wq, K_ext, V_ext, Wo):
    my = lax.axis_index("i")

    K = lax.dynamic_slice_in_dim(K_ext, my * H_PER, H_PER, axis=2)
    V = lax.dynamic_slice_in_dim(V_ext, my * H_PER, H_PER, axis=2)
    K = jnp.transpose(K, (0, 2, 1, 3)).astype(jnp.bfloat16)
    V = jnp.transpose(V, (0, 2, 1, 3)).astype(jnp.bfloat16)

    xb = x.astype(jnp.bfloat16)
    Wqb = (Wq * 0.125).astype(jnp.bfloat16)
    Wob = Wo.astype(jnp.bfloat16)

    def body(x_ref, wq_ref, k_ref, v_ref, wo_ref, out_ref,
             pbuf, rs_buf, ag_buf, red_ref, redb_ref,
             ss1, rs1, ss2, rs2):
        pos = lax.axis_index("i")

        barrier_sem = pltpu.get_barrier_semaphore()
        for q in range(1, N_DEV):
            pl.semaphore_signal(
                barrier_sem, inc=1,
                device_id=((pos + q) % N_DEV,),
                device_id_type=pl.DeviceIdType.MESH,
            )
        pl.semaphore_wait(barrier_sem, N_DEV - 1)

        qi = lax.broadcasted_iota(jnp.int32, (SQ, SKV), 0)
        ki = lax.broadcasted_iota(jnp.int32, (SQ, SKV), 1)
        maskbias = jnp.where(jnp.abs(qi - ki) <= WINDOW,
                             0.0, -1e9).astype(jnp.bfloat16)

        def send_chunk(c):
            return pltpu.make_async_remote_copy(
                src_ref=pbuf.at[pl.ds(c * CHUNK, CHUNK), :],
                dst_ref=rs_buf.at[pos],
                send_sem=ss1.at[c],
                recv_sem=rs1.at[pos],
                device_id=(c,),
                device_id_type=pl.DeviceIdType.MESH,
            )

        maskbias_f = maskbias.astype(jnp.float32)
        for b in range(B):
            q = jnp.dot(x_ref[b], wq_ref[...],
                        preferred_element_type=jnp.float32)
            qb = q.astype(jnp.bfloat16)
            ctx_cols = []
            for h in range(H_PER):
                s = lax.dot_general(
                    qb[:, DH * h:DH * (h + 1)], k_ref[b, h],
                    (((1,), (1,)), ((), ())),
                    preferred_element_type=jnp.float32)
                w = jnp.exp(s + maskbias_f)
                denom = jnp.sum(w, axis=-1, keepdims=True)
                wn = (w * (1.0 / denom)).astype(jnp.bfloat16)
                ctx_cols.append(
                    jnp.dot(wn, v_ref[b, h],
                            preferred_element_type=jnp.float32))
            ctx = jnp.concatenate(ctx_cols, axis=1).astype(jnp.bfloat16)
            partial = jnp.dot(ctx, wo_ref[...],
                              preferred_element_type=jnp.float32)
            pbuf[SQ * b:SQ * (b + 1), :] = partial.astype(jnp.bfloat16)
            for c in range(CPB * b, CPB * (b + 1)):
                @pl.when(c != pos)
                def _(c=c):
                    send_chunk(c).start()

        red_ref[...] = pbuf[pl.ds(pos * CHUNK, CHUNK), :].astype(jnp.float32)
        for s in range(N_DEV):
            @pl.when(s != pos)
            def _(s=s):
                recv = pltpu.make_async_remote_copy(
                    src_ref=pbuf.at[pl.ds(0, CHUNK), :],
                    dst_ref=rs_buf.at[s],
                    send_sem=ss1.at[s],
                    recv_sem=rs1.at[s],
                    device_id=(s,),
                    device_id_type=pl.DeviceIdType.MESH,
                )
                recv.wait_recv()
                red_ref[...] += rs_buf[s].astype(jnp.float32)
        redb_ref[...] = red_ref[...].astype(jnp.bfloat16)

        for t in range(N_DEV):
            @pl.when(t != pos)
            def _(t=t):
                pltpu.make_async_remote_copy(
                    src_ref=redb_ref,
                    dst_ref=ag_buf.at[pos],
                    send_sem=ss2.at[t],
                    recv_sem=rs2.at[pos],
                    device_id=(t,),
                    device_id_type=pl.DeviceIdType.MESH,
                ).start()

        for c in range(N_DEV):
            @pl.when(c != pos)
            def _(c=c):
                send_chunk(c).wait_send()

        out_ref[pl.ds(pos * CHUNK, CHUNK), :] = red_ref[...]

        for s in range(N_DEV):
            @pl.when(s != pos)
            def _(s=s):
                recv = pltpu.make_async_remote_copy(
                    src_ref=redb_ref,
                    dst_ref=ag_buf.at[s],
                    send_sem=ss2.at[s],
                    recv_sem=rs2.at[s],
                    device_id=(s,),
                    device_id_type=pl.DeviceIdType.MESH,
                )
                recv.wait_recv()
                out_ref[s * CHUNK:(s + 1) * CHUNK, :] = (
                    ag_buf[s].astype(jnp.float32))
        for t in range(N_DEV):
            @pl.when(t != pos)
            def _(t=t):
                pltpu.make_async_remote_copy(
                    src_ref=redb_ref,
                    dst_ref=ag_buf.at[pos],
                    send_sem=ss2.at[t],
                    recv_sem=rs2.at[pos],
                    device_id=(t,),
                    device_id_type=pl.DeviceIdType.MESH,
                ).wait_send()

    out_flat = pl.pallas_call(
        body,
        out_shape=jax.ShapeDtypeStruct((ROWS, D_MODEL), jnp.float32),
        in_specs=[pl.BlockSpec(memory_space=pltpu.VMEM)] * 5,
        out_specs=pl.BlockSpec(memory_space=pltpu.VMEM),
        scratch_shapes=[
            pltpu.VMEM((ROWS, D_MODEL), jnp.bfloat16),
            pltpu.VMEM((N_DEV, CHUNK, D_MODEL), jnp.bfloat16),
            pltpu.VMEM((N_DEV, CHUNK, D_MODEL), jnp.bfloat16),
            pltpu.VMEM((CHUNK, D_MODEL), jnp.float32),
            pltpu.VMEM((CHUNK, D_MODEL), jnp.bfloat16),
            pltpu.SemaphoreType.DMA((N_DEV,)),
            pltpu.SemaphoreType.DMA((N_DEV,)),
            pltpu.SemaphoreType.DMA((N_DEV,)),
            pltpu.SemaphoreType.DMA((N_DEV,)),
        ],
        compiler_params=pltpu.CompilerParams(collective_id=0),
    )(xb, Wqb, K, V, Wob)
    return out_flat.reshape(B, SQ, D_MODEL)
